# baseline (speedup 1.0000x reference)
import functools

import jax
import jax.numpy as jnp
from jax import lax
from jax.experimental import pallas as pl
from jax.experimental.pallas import tpu as pltpu
from jax.experimental.pallas import tpu_sc as plsc

N = 10000
E = 320000
EL = 100000
D = 128
HID = 256
QW = 64
OUT = 16

NS = 16
NC = 2
CHUNK = 128
EPT = 20480
NCH = EPT // CHUNK
HNCH = NCH // 2
EPAD = NS * EPT
ELCH = 25
ELPAD = NC * NS * ELCH * CHUNK
ROWS_PT = 640
ROWS_LAST = N - 15 * ROWS_PT

_f32 = jnp.float32
_i32 = jnp.int32


def _mesh():
    return plsc.VectorSubcoreMesh(core_axis_name="c", subcore_axis_name="s")


def _sc_params():
    return pltpu.CompilerParams(needs_layout_passes=False,
                                use_tc_tiling_on_sc=False)


def _zero_vmem_2d(buf, width):
    zero16 = lax.full((16,), 0.0, _f32)

    def row(r, carry):
        for cc in range(width // 16):
            buf[r, pl.ds(cc * 16, 16)] = zero16
        return carry

    lax.fori_loop(0, CHUNK, row, 0)


def _zero_spmem_rows(zbuf, agg_sp, sid):
    @pl.when(sid < 15)
    def _():
        for k in range(5):
            pltpu.sync_copy(zbuf,
                            agg_sp.at[pl.ds(sid * ROWS_PT + k * CHUNK,
                                            CHUNK)])

    @pl.when(sid == 15)
    def _():
        for k in range(3):
            pltpu.sync_copy(zbuf,
                            agg_sp.at[pl.ds(15 * ROWS_PT + k * CHUNK,
                                            CHUNK)])
        pltpu.sync_copy(zbuf.at[pl.ds(0, 16)],
                        agg_sp.at[pl.ds(15 * ROWS_PT + 3 * CHUNK, 16)])


def _spmem_to_hbm_rows(agg_sp, out_hbm, bounce, sid):
    @pl.when(sid < 15)
    def _():
        for k in range(5):
            rows = pl.ds(sid * ROWS_PT + k * CHUNK, CHUNK)
            pltpu.sync_copy(agg_sp.at[rows], bounce)
            pltpu.sync_copy(bounce, out_hbm.at[rows])

    @pl.when(sid == 15)
    def _():
        for k in range(3):
            rows = pl.ds(15 * ROWS_PT + k * CHUNK, CHUNK)
            pltpu.sync_copy(agg_sp.at[rows], bounce)
            pltpu.sync_copy(bounce, out_hbm.at[rows])
        rows = pl.ds(15 * ROWS_PT + 3 * CHUNK, 16)
        pltpu.sync_copy(agg_sp.at[rows], bounce.at[pl.ds(0, 16)])
        pltpu.sync_copy(bounce.at[pl.ds(0, 16)], out_hbm.at[rows])


def _zero_denom(p_b, denom_sp, sid):
    zero16 = lax.full((16,), 0.0, _f32)
    for i in range(8):
        p_b[pl.ds(i * 16, 16)] = zero16

    @pl.when(sid < 10)
    def _():
        for k in range(8):
            pltpu.sync_copy(p_b.at[pl.ds(0, 120)],
                            denom_sp.at[pl.ds(sid * 1000 + k * 120, 120)])
        pltpu.sync_copy(p_b.at[pl.ds(0, 40)],
                        denom_sp.at[pl.ds(sid * 1000 + 960, 40)])


def _edge_p(s_score_v, d_score_v, src3h, dst3h, sid, cl, cg, g, iota16):
    s_i = src3h[cl, pl.ds(g * 16, 16)]
    d_i = dst3h[cl, pl.ds(g * 16, 16)]
    s_v = plsc.load_gather(s_score_v, [s_i])
    d_v = plsc.load_gather(d_score_v, [d_i])
    l = s_v + d_v
    l = jnp.where(l > 0, l, 0.2 * l)
    p = jnp.exp(l)
    gid = sid * EPT + cg * CHUNK + g * 16 + iota16
    return jnp.where(gid < E, p, 0.0), d_i


def _load_idx_half(src3_hbm, dst3_hbm, sid, h0, src3h, dst3h):
    pltpu.sync_copy(src3_hbm.at[sid, pl.ds(h0, HNCH)], src3h)
    pltpu.sync_copy(dst3_hbm.at[sid, pl.ds(h0, HNCH)], dst3h)


def _sc_denom_phase(s_score_v, d_score_v, src3_hbm, dst3_hbm, src3h, dst3h,
                    p_bufs, p_sems, denom_sp, sid, iota16):
    zero16 = lax.full((16,), 0.0, _f32)
    for h0 in (0, HNCH):
        _load_idx_half(src3_hbm, dst3_hbm, sid, h0, src3h, dst3h)
        for par in (0, 1):
            for g in range(8):
                p_bufs[par][pl.ds(g * 16, 16)] = zero16
            pltpu.async_copy(p_bufs[par], denom_sp.at[dst3h.at[par]],
                             p_sems[par], add=True)

        def pair(i, carry):
            for par in (0, 1):
                cl = 2 * i + par
                pb = p_bufs[par]
                pltpu.make_async_copy(pb, denom_sp.at[dst3h.at[cl]],
                                      p_sems[par]).wait()
                for g in range(8):
                    p, _ = _edge_p(s_score_v, d_score_v, src3h, dst3h, sid,
                                   cl, h0 + cl, g, iota16)
                    pb[pl.ds(g * 16, 16)] = p
                pltpu.async_copy(pb, denom_sp.at[dst3h.at[cl]], p_sems[par],
                                 add=True)
            return carry

        lax.fori_loop(0, HNCH // 2, pair, 0)
        for par in (0, 1):
            pltpu.make_async_copy(p_bufs[par], denom_sp.at[dst3h.at[par]],
                                  p_sems[par]).wait()


def _sc_aggregate(h_hbm, agg_sp, denom_v, s_score_v, d_score_v,
                  src3_hbm, dst3_hbm, src3h, dst3h, g_bufs,
                  s_bufs, g_sems, s_sems, width, sid, iota16,
                  halves):
    nsub = width // 16
    lane_consts = [lax.full((16,), j, _i32) for j in range(16)]

    def scale_chunk(cl, cg, par):
        g_buf, s_buf = g_bufs[par], s_bufs[par]

        def grp(g, carry2):
            p, d_i = _edge_p(s_score_v, d_score_v, src3h, dst3h, sid, cl,
                             cg, g, iota16)
            den16 = plsc.load_gather(denom_v, [d_i])
            alpha16 = p / (den16 + 1e-16)
            for j in range(16):
                a_b = alpha16.at[lane_consts[j]].get(
                    mode="promise_in_bounds")
                r = g * 16 + j
                for cc in range(nsub):
                    s_buf[r, pl.ds(cc * 16, 16)] = (
                        g_buf[r, pl.ds(cc * 16, 16)] * a_b)
            return carry2

        lax.fori_loop(0, 8, grp, 0)

    def issue_gather(cl, par):
        pltpu.async_copy(h_hbm.at[src3h.at[cl]], g_bufs[par], g_sems[par])

    def wait_gather(cl, par):
        pltpu.make_async_copy(h_hbm.at[src3h.at[cl]], g_bufs[par],
                              g_sems[par]).wait()

    def issue_scatter(cl, par):
        pltpu.async_copy(s_bufs[par], agg_sp.at[dst3h.at[cl]], s_sems[par],
                         add=True)

    def wait_scatter(cl, par):
        pltpu.make_async_copy(s_bufs[par], agg_sp.at[dst3h.at[cl]],
                              s_sems[par]).wait()

    for h0 in halves:
        _load_idx_half(src3_hbm, dst3_hbm, sid, h0, src3h, dst3h)
        _zero_vmem_2d(s_bufs[0], width)
        _zero_vmem_2d(s_bufs[1], width)
        issue_scatter(0, 0)
        issue_scatter(1, 1)
        issue_gather(0, 0)

        def pair(i, carry):
            for par in (0, 1):
                cl = 2 * i + par
                issue_gather(jnp.minimum(cl + 1, HNCH - 1), 1 - par)
                wait_gather(cl, par)
                wait_scatter(cl, par)
                scale_chunk(cl, h0 + cl, par)
                issue_scatter(cl, par)
            return carry

        lax.fori_loop(0, HNCH // 2, pair, 0)
        wait_gather(HNCH - 1, 0)
        wait_scatter(HNCH - 2, 0)
        wait_scatter(HNCH - 1, 1)


def _make_sc_agg_wide():
    scratch = [
        pltpu.VMEM((N,), _f32),
        pltpu.VMEM((N,), _f32),
        pltpu.VMEM((HNCH, CHUNK), _i32),
        pltpu.VMEM((HNCH, CHUNK), _i32),
        pltpu.VMEM((CHUNK,), _f32),
        pltpu.VMEM((CHUNK,), _f32),
        pltpu.VMEM((N,), _f32),
        pltpu.VMEM((CHUNK, QW), _f32),
        pltpu.VMEM((CHUNK, QW), _f32),
        pltpu.VMEM((CHUNK, QW), _f32),
        pltpu.VMEM((CHUNK, QW), _f32),
    ] + [pltpu.SemaphoreType.DMA] * 6 + [
        pltpu.VMEM_SHARED((N,), _f32),
        pltpu.VMEM_SHARED((N, QW), _f32),
    ]

    @functools.partial(
        pl.kernel,
        out_type=tuple(jax.ShapeDtypeStruct((N, QW), _f32)
                       for _ in range(4)),
        mesh=_mesh(),
        scratch_types=scratch,
        compiler_params=_sc_params(),
    )
    def k(h0_hbm, h1_hbm, h2_hbm, h3_hbm, s_hbm, d_hbm, src3_hbm, dst3_hbm,
          a0_hbm, a1_hbm, a2_hbm, a3_hbm,
          s_score_v, d_score_v, src3h, dst3h, p_b0, p_b1, denom_v,
          g0, g1, s0, s1, sem_p0, sem_p1, sem_g0, sem_g1, sem_s0, sem_s1,
          denom_sp, agg_sp):
        cid = lax.axis_index("c")
        sid = lax.axis_index("s")
        iota16 = lax.iota(_i32, 16)

        pltpu.sync_copy(s_hbm, s_score_v)
        pltpu.sync_copy(d_hbm, d_score_v)

        _zero_vmem_2d(s0, QW)
        _zero_spmem_rows(s0, agg_sp, sid)
        _zero_denom(p_b0, denom_sp, sid)
        plsc.subcore_barrier()

        _sc_denom_phase(s_score_v, d_score_v, src3_hbm, dst3_hbm, src3h,
                        dst3h, (p_b0, p_b1), (sem_p0, sem_p1), denom_sp,
                        sid, iota16)
        plsc.subcore_barrier()
        pltpu.sync_copy(denom_sp, denom_v)

        def one_pass(h_hbm, out_hbm):
            _sc_aggregate(h_hbm, agg_sp, denom_v, s_score_v, d_score_v,
                          src3_hbm, dst3_hbm, src3h, dst3h,
                          (g0, g1), (s0, s1), (sem_g0, sem_g1),
                          (sem_s0, sem_s1), QW, sid, iota16, (0, HNCH))
            plsc.subcore_barrier()
            _spmem_to_hbm_rows(agg_sp, out_hbm, g0, sid)

        def second_pass_prep():
            plsc.subcore_barrier()
            _zero_vmem_2d(s0, QW)
            _zero_spmem_rows(s0, agg_sp, sid)
            plsc.subcore_barrier()

        @pl.when(cid == 0)
        def _():
            one_pass(h0_hbm, a0_hbm)
            second_pass_prep()
            one_pass(h1_hbm, a1_hbm)

        @pl.when(cid == 1)
        def _():
            one_pass(h2_hbm, a2_hbm)
            second_pass_prep()
            one_pass(h3_hbm, a3_hbm)

    return k


def _make_sc_agg_small():
    scratch = [
        pltpu.VMEM((N,), _f32),
        pltpu.VMEM((N,), _f32),
        pltpu.VMEM((HNCH, CHUNK), _i32),
        pltpu.VMEM((HNCH, CHUNK), _i32),
        pltpu.VMEM((CHUNK,), _f32),
        pltpu.VMEM((CHUNK,), _f32),
        pltpu.VMEM((N,), _f32),
        pltpu.VMEM((CHUNK, OUT), _f32),
        pltpu.VMEM((CHUNK, OUT), _f32),
        pltpu.VMEM((CHUNK, OUT), _f32),
        pltpu.VMEM((CHUNK, OUT), _f32),
    ] + [pltpu.SemaphoreType.DMA] * 6 + [
        pltpu.VMEM_SHARED((N,), _f32),
        pltpu.VMEM_SHARED((N, OUT), _f32),
    ]

    @functools.partial(
        pl.kernel,
        out_type=jax.ShapeDtypeStruct((NC, N, OUT), _f32),
        mesh=_mesh(),
        scratch_types=scratch,
        compiler_params=_sc_params(),
    )
    def k(h3_hbm, s_hbm, d_hbm, src3_hbm, dst3_hbm, part_hbm,
          s_score_v, d_score_v, src3h, dst3h, p_b0, p_b1, denom_v,
          g0, g1, s0, s1, sem_p0, sem_p1, sem_g0, sem_g1, sem_s0, sem_s1,
          denom_sp, agg_sp):
        cid = lax.axis_index("c")
        sid = lax.axis_index("s")
        iota16 = lax.iota(_i32, 16)

        pltpu.sync_copy(s_hbm, s_score_v)
        pltpu.sync_copy(d_hbm, d_score_v)

        _zero_vmem_2d(s0, OUT)
        _zero_spmem_rows(s0, agg_sp, sid)
        _zero_denom(p_b0, denom_sp, sid)
        plsc.subcore_barrier()

        _sc_denom_phase(s_score_v, d_score_v, src3_hbm, dst3_hbm, src3h,
                        dst3h, (p_b0, p_b1), (sem_p0, sem_p1), denom_sp,
                        sid, iota16)
        plsc.subcore_barrier()
        pltpu.sync_copy(denom_sp, denom_v)

        _sc_aggregate(h3_hbm, agg_sp, denom_v, s_score_v, d_score_v,
                      src3_hbm, dst3_hbm, src3h, dst3h,
                      (g0, g1), (s0, s1), (sem_g0, sem_g1),
                      (sem_s0, sem_s1), OUT, sid, iota16,
                      (cid * HNCH,))

        plsc.subcore_barrier()
        _spmem_to_hbm_rows(agg_sp, part_hbm.at[cid], g0, sid)

    return k


def _make_sc_classifier():
    scratch = [
        pltpu.VMEM((N,), _f32),
        pltpu.VMEM((N,), _f32),
        pltpu.VMEM((ELCH, CHUNK), _i32),
        pltpu.VMEM((ELCH, CHUNK), _i32),
        pltpu.VMEM((ELCH, CHUNK), _f32),
    ]

    @functools.partial(
        pl.kernel,
        out_type=jax.ShapeDtypeStruct((NC * NS, ELCH, CHUNK), _f32),
        mesh=_mesh(),
        scratch_types=scratch,
        compiler_params=_sc_params(),
    )
    def k(u_hbm, v_hbm, s3_hbm, d3_hbm, pred_hbm, u_v, v_v, s_v, d_v, out_v):
        cid = lax.axis_index("c")
        sid = lax.axis_index("s")
        wid = sid * NC + cid

        pltpu.sync_copy(u_hbm, u_v)
        pltpu.sync_copy(v_hbm, v_v)
        pltpu.sync_copy(s3_hbm.at[wid], s_v)
        pltpu.sync_copy(d3_hbm.at[wid], d_v)

        def chunk(c, carry):
            for g in range(8):
                s_i = s_v[c, pl.ds(g * 16, 16)]
                d_i = d_v[c, pl.ds(g * 16, 16)]
                u = plsc.load_gather(u_v, [s_i])
                v = plsc.load_gather(v_v, [d_i])
                z = u + v
                out_v[c, pl.ds(g * 16, 16)] = 1.0 / (1.0 + jnp.exp(-z))
            return carry

        lax.fori_loop(0, ELCH, chunk, 0)
        pltpu.sync_copy(out_v, pred_hbm.at[wid])

    return k


BLK = 1000
GRID = N // BLK


def _tc_layer1(x, W1, A1):
    def body(x_ref, w_ref, a_ref, q0_ref, q1_ref, q2_ref, q3_ref, sc_ref):
        h = jnp.dot(x_ref[...], w_ref[...], preferred_element_type=_f32)
        for i, q in enumerate((q0_ref, q1_ref, q2_ref, q3_ref)):
            q[...] = h[:, i * QW:(i + 1) * QW]
        sc_ref[...] = jnp.dot(h, a_ref[...], preferred_element_type=_f32)

    qspec = pl.BlockSpec((BLK, QW), lambda i: (i, 0))
    return pl.pallas_call(
        body,
        grid=(GRID,),
        in_specs=[
            pl.BlockSpec((BLK, D), lambda i: (i, 0)),
            pl.BlockSpec((D, HID), lambda i: (0, 0)),
            pl.BlockSpec((HID, 2), lambda i: (0, 0)),
        ],
        out_specs=[qspec, qspec, qspec, qspec,
                   pl.BlockSpec((BLK, 2), lambda i: (i, 0))],
        out_shape=[jax.ShapeDtypeStruct((N, QW), _f32) for _ in range(4)]
        + [jax.ShapeDtypeStruct((N, 2), _f32)],
    )(x, W1, A1)


def _tc_layer_mid(quarters, b4, W, A, n_out, split):
    def body(q0_ref, q1_ref, q2_ref, q3_ref, b_ref, w_ref, a_ref, *outs):
        w = w_ref[...]
        h = None
        for i, q in enumerate((q0_ref, q1_ref, q2_ref, q3_ref)):
            xq = jax.nn.relu(q[...] + b_ref[i, :])
            part = jnp.dot(xq, w[i * QW:(i + 1) * QW, :],
                           preferred_element_type=_f32)
            h = part if h is None else h + part
        if split:
            for i in range(4):
                outs[i][...] = h[:, i * QW:(i + 1) * QW]
        else:
            outs[0][...] = h
        outs[-1][...] = jnp.dot(h, a_ref[...], preferred_element_type=_f32)

    qspec = pl.BlockSpec((BLK, QW), lambda i: (i, 0))
    scspec = pl.BlockSpec((BLK, 2), lambda i: (i, 0))
    if split:
        out_specs = [qspec, qspec, qspec, qspec, scspec]
        out_shape = [jax.ShapeDtypeStruct((N, QW), _f32) for _ in range(4)] \
            + [jax.ShapeDtypeStruct((N, 2), _f32)]
    else:
        out_specs = [pl.BlockSpec((BLK, n_out), lambda i: (i, 0)), scspec]
        out_shape = [jax.ShapeDtypeStruct((N, n_out), _f32),
                     jax.ShapeDtypeStruct((N, 2), _f32)]

    return pl.pallas_call(
        body,
        grid=(GRID,),
        in_specs=[qspec, qspec, qspec, qspec,
                  pl.BlockSpec((4, QW), lambda i: (0, 0)),
                  pl.BlockSpec((HID, n_out), lambda i: (0, 0)),
                  pl.BlockSpec((n_out, 2), lambda i: (0, 0))],
        out_specs=out_specs,
        out_shape=out_shape,
    )(*quarters, b4, W, A)


def _tc_emb_uv(parts, b3_2d, Wuv, buv_2d):
    def body(p_ref, b_ref, w_ref, bu_ref, emb_ref, uv_ref):
        emb = p_ref[0] + p_ref[1] + b_ref[0, :]
        emb_ref[...] = emb
        uv_ref[...] = (jnp.dot(emb, w_ref[...], preferred_element_type=_f32)
                       + bu_ref[0, :])

    return pl.pallas_call(
        body,
        grid=(GRID,),
        in_specs=[
            pl.BlockSpec((NC, BLK, OUT), lambda i: (0, i, 0)),
            pl.BlockSpec((1, OUT), lambda i: (0, 0)),
            pl.BlockSpec((OUT, 2), lambda i: (0, 0)),
            pl.BlockSpec((1, 2), lambda i: (0, 0)),
        ],
        out_specs=[
            pl.BlockSpec((BLK, OUT), lambda i: (i, 0)),
            pl.BlockSpec((BLK, 2), lambda i: (i, 0)),
        ],
        out_shape=[
            jax.ShapeDtypeStruct((N, OUT), _f32),
            jax.ShapeDtypeStruct((N, 2), _f32),
        ],
    )(parts, b3_2d, Wuv, buv_2d)


_sc_agg_wide = functools.lru_cache(maxsize=None)(_make_sc_agg_wide)
_sc_agg_small = functools.lru_cache(maxsize=None)(_make_sc_agg_small)
_sc_classifier = functools.lru_cache(maxsize=None)(_make_sc_classifier)


def kernel(x, edge_index, edge_label_index, W1, as1, ad1, b1, W2, as2, ad2,
           b2, W3, as3, ad3, b3, Wc, bc):
    src = edge_index[0].astype(_i32)
    dst = edge_index[1].astype(_i32)
    pad = jnp.zeros((EPAD - E,), _i32)
    src3 = jnp.concatenate([src, pad]).reshape(NS, NCH, CHUNK)
    dst3 = jnp.concatenate([dst, pad]).reshape(NS, NCH, CHUNK)

    elpad = jnp.zeros((ELPAD - EL,), _i32)
    s3 = jnp.concatenate([edge_label_index[0].astype(_i32), elpad])
    d3 = jnp.concatenate([edge_label_index[1].astype(_i32), elpad])
    s3 = s3.reshape(NC * NS, ELCH, CHUNK)
    d3 = d3.reshape(NC * NS, ELCH, CHUNK)

    A1 = jnp.stack([as1, ad1], axis=1)
    A2 = jnp.stack([as2, ad2], axis=1)
    A3 = jnp.stack([as3, ad3], axis=1)
    b1_4 = b1.reshape(4, QW)
    b2_4 = b2.reshape(4, QW)
    b3_2d = b3.reshape(1, OUT)
    Wuv = Wc.reshape(2, OUT).T
    buv_2d = jnp.stack([bc[0], jnp.zeros((), _f32)]).reshape(1, 2)

    q0, q1, q2, q3, sc1 = _tc_layer1(x, W1, A1)
    a = _sc_agg_wide()(q0, q1, q2, q3, sc1[:, 0], sc1[:, 1], src3, dst3)

    *q2s, sc2 = _tc_layer_mid(a, b1_4, W2, A2, HID, True)
    a2 = _sc_agg_wide()(*q2s, sc2[:, 0], sc2[:, 1], src3, dst3)

    h3, sc3 = _tc_layer_mid(a2, b2_4, W3, A3, OUT, False)
    parts = _sc_agg_small()(h3, sc3[:, 0], sc3[:, 1], src3, dst3)

    emb, uv = _tc_emb_uv(parts, b3_2d, Wuv, buv_2d)
    pred3 = _sc_classifier()(uv[:, 0], uv[:, 1], s3, d3)
    pred = pred3.reshape(-1)[:EL]
    return pred, emb


# speedup vs baseline: 1.3897x; 1.3897x over previous
import functools

import jax
import jax.numpy as jnp
from jax import lax
from jax.experimental import pallas as pl
from jax.experimental.pallas import tpu as pltpu
from jax.experimental.pallas import tpu_sc as plsc

N = 10000
E = 320000
EL = 100000
D = 128
HID = 256
QW = 64
OUT = 16

NS = 16
NC = 2
CHUNK = 128
EPT = 20480
NCH = EPT // CHUNK
HNCH = NCH // 2
EPAD = NS * EPT
ELCH = 25
ELPAD = NC * NS * ELCH * CHUNK
ROWS_PT = 640
ROWS_LAST = N - 15 * ROWS_PT

_f32 = jnp.float32
_i32 = jnp.int32


def _mesh():
    return plsc.VectorSubcoreMesh(core_axis_name="c", subcore_axis_name="s")


def _sc_params():
    return pltpu.CompilerParams(needs_layout_passes=False,
                                use_tc_tiling_on_sc=False)


def _zero_vmem_2d(buf, width):
    zero16 = lax.full((16,), 0.0, _f32)

    def row(r, carry):
        for cc in range(width // 16):
            buf[r, pl.ds(cc * 16, 16)] = zero16
        return carry

    lax.fori_loop(0, CHUNK, row, 0)


def _zero_spmem_rows(zbuf, agg_sp, sid):
    @pl.when(sid < 15)
    def _():
        for k in range(5):
            pltpu.sync_copy(zbuf,
                            agg_sp.at[pl.ds(sid * ROWS_PT + k * CHUNK,
                                            CHUNK)])

    @pl.when(sid == 15)
    def _():
        for k in range(3):
            pltpu.sync_copy(zbuf,
                            agg_sp.at[pl.ds(15 * ROWS_PT + k * CHUNK,
                                            CHUNK)])
        pltpu.sync_copy(zbuf.at[pl.ds(0, 16)],
                        agg_sp.at[pl.ds(15 * ROWS_PT + 3 * CHUNK, 16)])


def _spmem_to_hbm_rows(agg_sp, out_hbm, bounce, sid):
    @pl.when(sid < 15)
    def _():
        for k in range(5):
            rows = pl.ds(sid * ROWS_PT + k * CHUNK, CHUNK)
            pltpu.sync_copy(agg_sp.at[rows], bounce)
            pltpu.sync_copy(bounce, out_hbm.at[rows])

    @pl.when(sid == 15)
    def _():
        for k in range(3):
            rows = pl.ds(15 * ROWS_PT + k * CHUNK, CHUNK)
            pltpu.sync_copy(agg_sp.at[rows], bounce)
            pltpu.sync_copy(bounce, out_hbm.at[rows])
        rows = pl.ds(15 * ROWS_PT + 3 * CHUNK, 16)
        pltpu.sync_copy(agg_sp.at[rows], bounce.at[pl.ds(0, 16)])
        pltpu.sync_copy(bounce.at[pl.ds(0, 16)], out_hbm.at[rows])


def _zero_denom(p_b, denom_sp, sid):
    zero16 = lax.full((16,), 0.0, _f32)
    for i in range(8):
        p_b[pl.ds(i * 16, 16)] = zero16

    @pl.when(sid < 10)
    def _():
        for k in range(8):
            pltpu.sync_copy(p_b.at[pl.ds(0, 120)],
                            denom_sp.at[pl.ds(sid * 1000 + k * 120, 120)])
        pltpu.sync_copy(p_b.at[pl.ds(0, 40)],
                        denom_sp.at[pl.ds(sid * 1000 + 960, 40)])


def _edge_p(s_score_v, d_score_v, src3h, dst3h, sid, cl, cg, g, iota16):
    s_i = src3h[cl, pl.ds(g * 16, 16)]
    d_i = dst3h[cl, pl.ds(g * 16, 16)]
    s_v = plsc.load_gather(s_score_v, [s_i])
    d_v = plsc.load_gather(d_score_v, [d_i])
    l = s_v + d_v
    l = jnp.where(l > 0, l, 0.2 * l)
    p = jnp.exp(l)
    gid = sid * EPT + cg * CHUNK + g * 16 + iota16
    return jnp.where(gid < E, p, 0.0), d_i


def _load_idx_half(src3_hbm, dst3_hbm, sid, h0, src3h, dst3h):
    pltpu.sync_copy(src3_hbm.at[sid, pl.ds(h0, HNCH)], src3h)
    pltpu.sync_copy(dst3_hbm.at[sid, pl.ds(h0, HNCH)], dst3h)


def _sc_denom_phase(s_score_v, d_score_v, src3_hbm, dst3_hbm, src3h, dst3h,
                    p_bufs, p_sems, denom_sp, sid, iota16):
    zero16 = lax.full((16,), 0.0, _f32)
    for h0 in (0, HNCH):
        _load_idx_half(src3_hbm, dst3_hbm, sid, h0, src3h, dst3h)
        for par in (0, 1):
            for g in range(8):
                p_bufs[par][pl.ds(g * 16, 16)] = zero16
            pltpu.async_copy(p_bufs[par], denom_sp.at[dst3h.at[par]],
                             p_sems[par], add=True)

        def pair(i, carry):
            for par in (0, 1):
                cl = 2 * i + par
                pb = p_bufs[par]
                pltpu.make_async_copy(pb, denom_sp.at[dst3h.at[cl]],
                                      p_sems[par]).wait()
                for g in range(8):
                    p, _ = _edge_p(s_score_v, d_score_v, src3h, dst3h, sid,
                                   cl, h0 + cl, g, iota16)
                    pb[pl.ds(g * 16, 16)] = p
                pltpu.async_copy(pb, denom_sp.at[dst3h.at[cl]], p_sems[par],
                                 add=True)
            return carry

        lax.fori_loop(0, HNCH // 2, pair, 0)
        for par in (0, 1):
            pltpu.make_async_copy(p_bufs[par], denom_sp.at[dst3h.at[par]],
                                  p_sems[par]).wait()


def _sc_aggregate(h_hbm, agg_sp, denom_v, s_score_v, d_score_v,
                  src3_hbm, dst3_hbm, src3h, dst3h, g_bufs,
                  s_bufs, g_sems, s_sems, width, sid, iota16,
                  halves):
    nsub = width // 16
    lane_consts = [lax.full((16,), j, _i32) for j in range(16)]

    def scale_chunk(cl, cg, par):
        g_buf, s_buf = g_bufs[par], s_bufs[par]

        def grp(g, carry2):
            p, d_i = _edge_p(s_score_v, d_score_v, src3h, dst3h, sid, cl,
                             cg, g, iota16)
            den16 = plsc.load_gather(denom_v, [d_i])
            alpha16 = p / (den16 + 1e-16)
            for j in range(16):
                a_b = alpha16.at[lane_consts[j]].get(
                    mode="promise_in_bounds")
                r = g * 16 + j
                for cc in range(nsub):
                    s_buf[r, pl.ds(cc * 16, 16)] = (
                        g_buf[r, pl.ds(cc * 16, 16)] * a_b)
            return carry2

        lax.fori_loop(0, 8, grp, 0, unroll=2)

    def issue_gather(cl, par):
        pltpu.async_copy(h_hbm.at[src3h.at[cl]], g_bufs[par], g_sems[par])

    def wait_gather(cl, par):
        pltpu.make_async_copy(h_hbm.at[src3h.at[cl]], g_bufs[par],
                              g_sems[par]).wait()

    def issue_scatter(cl, par):
        pltpu.async_copy(s_bufs[par], agg_sp.at[dst3h.at[cl]], s_sems[par],
                         add=True)

    def wait_scatter(cl, par):
        pltpu.make_async_copy(s_bufs[par], agg_sp.at[dst3h.at[cl]],
                              s_sems[par]).wait()

    for h0 in halves:
        _load_idx_half(src3_hbm, dst3_hbm, sid, h0, src3h, dst3h)
        _zero_vmem_2d(s_bufs[0], width)
        _zero_vmem_2d(s_bufs[1], width)
        issue_scatter(0, 0)
        issue_scatter(1, 1)
        issue_gather(0, 0)

        def pair(i, carry):
            for par in (0, 1):
                cl = 2 * i + par
                issue_gather(jnp.minimum(cl + 1, HNCH - 1), 1 - par)
                wait_gather(cl, par)
                wait_scatter(cl, par)
                scale_chunk(cl, h0 + cl, par)
                issue_scatter(cl, par)
            return carry

        lax.fori_loop(0, HNCH // 2, pair, 0)
        wait_gather(HNCH - 1, 0)
        wait_scatter(HNCH - 2, 0)
        wait_scatter(HNCH - 1, 1)


def _make_sc_agg_wide():
    scratch = [
        pltpu.VMEM((N,), _f32),
        pltpu.VMEM((N,), _f32),
        pltpu.VMEM((HNCH, CHUNK), _i32),
        pltpu.VMEM((HNCH, CHUNK), _i32),
        pltpu.VMEM((CHUNK,), _f32),
        pltpu.VMEM((CHUNK,), _f32),
        pltpu.VMEM((N,), _f32),
        pltpu.VMEM((CHUNK, QW), _f32),
        pltpu.VMEM((CHUNK, QW), _f32),
        pltpu.VMEM((CHUNK, QW), _f32),
        pltpu.VMEM((CHUNK, QW), _f32),
    ] + [pltpu.SemaphoreType.DMA] * 6 + [
        pltpu.VMEM_SHARED((N,), _f32),
        pltpu.VMEM_SHARED((N, QW), _f32),
    ]

    @functools.partial(
        pl.kernel,
        out_type=tuple(jax.ShapeDtypeStruct((N, QW), _f32)
                       for _ in range(4)),
        mesh=_mesh(),
        scratch_types=scratch,
        compiler_params=_sc_params(),
    )
    def k(h0_hbm, h1_hbm, h2_hbm, h3_hbm, s_hbm, d_hbm, src3_hbm, dst3_hbm,
          a0_hbm, a1_hbm, a2_hbm, a3_hbm,
          s_score_v, d_score_v, src3h, dst3h, p_b0, p_b1, denom_v,
          g0, g1, s0, s1, sem_p0, sem_p1, sem_g0, sem_g1, sem_s0, sem_s1,
          denom_sp, agg_sp):
        cid = lax.axis_index("c")
        sid = lax.axis_index("s")
        iota16 = lax.iota(_i32, 16)

        pltpu.sync_copy(s_hbm, s_score_v)
        pltpu.sync_copy(d_hbm, d_score_v)

        _zero_vmem_2d(s0, QW)
        _zero_spmem_rows(s0, agg_sp, sid)
        _zero_denom(p_b0, denom_sp, sid)
        plsc.subcore_barrier()

        _sc_denom_phase(s_score_v, d_score_v, src3_hbm, dst3_hbm, src3h,
                        dst3h, (p_b0, p_b1), (sem_p0, sem_p1), denom_sp,
                        sid, iota16)
        plsc.subcore_barrier()
        pltpu.sync_copy(denom_sp, denom_v)

        def one_pass(h_hbm, out_hbm):
            _sc_aggregate(h_hbm, agg_sp, denom_v, s_score_v, d_score_v,
                          src3_hbm, dst3_hbm, src3h, dst3h,
                          (g0, g1), (s0, s1), (sem_g0, sem_g1),
                          (sem_s0, sem_s1), QW, sid, iota16, (0, HNCH))
            plsc.subcore_barrier()
            _spmem_to_hbm_rows(agg_sp, out_hbm, g0, sid)

        def second_pass_prep():
            plsc.subcore_barrier()
            _zero_vmem_2d(s0, QW)
            _zero_spmem_rows(s0, agg_sp, sid)
            plsc.subcore_barrier()

        @pl.when(cid == 0)
        def _():
            one_pass(h0_hbm, a0_hbm)
            second_pass_prep()
            one_pass(h1_hbm, a1_hbm)

        @pl.when(cid == 1)
        def _():
            one_pass(h2_hbm, a2_hbm)
            second_pass_prep()
            one_pass(h3_hbm, a3_hbm)

    return k


def _make_sc_agg_small():
    scratch = [
        pltpu.VMEM((N,), _f32),
        pltpu.VMEM((N,), _f32),
        pltpu.VMEM((HNCH, CHUNK), _i32),
        pltpu.VMEM((HNCH, CHUNK), _i32),
        pltpu.VMEM((CHUNK,), _f32),
        pltpu.VMEM((CHUNK,), _f32),
        pltpu.VMEM((N,), _f32),
        pltpu.VMEM((CHUNK, OUT), _f32),
        pltpu.VMEM((CHUNK, OUT), _f32),
        pltpu.VMEM((CHUNK, OUT), _f32),
        pltpu.VMEM((CHUNK, OUT), _f32),
    ] + [pltpu.SemaphoreType.DMA] * 6 + [
        pltpu.VMEM_SHARED((N,), _f32),
        pltpu.VMEM_SHARED((N, OUT), _f32),
    ]

    @functools.partial(
        pl.kernel,
        out_type=jax.ShapeDtypeStruct((NC, N, OUT), _f32),
        mesh=_mesh(),
        scratch_types=scratch,
        compiler_params=_sc_params(),
    )
    def k(h3_hbm, s_hbm, d_hbm, src3_hbm, dst3_hbm, part_hbm,
          s_score_v, d_score_v, src3h, dst3h, p_b0, p_b1, denom_v,
          g0, g1, s0, s1, sem_p0, sem_p1, sem_g0, sem_g1, sem_s0, sem_s1,
          denom_sp, agg_sp):
        cid = lax.axis_index("c")
        sid = lax.axis_index("s")
        iota16 = lax.iota(_i32, 16)

        pltpu.sync_copy(s_hbm, s_score_v)
        pltpu.sync_copy(d_hbm, d_score_v)

        _zero_vmem_2d(s0, OUT)
        _zero_spmem_rows(s0, agg_sp, sid)
        _zero_denom(p_b0, denom_sp, sid)
        plsc.subcore_barrier()

        _sc_denom_phase(s_score_v, d_score_v, src3_hbm, dst3_hbm, src3h,
                        dst3h, (p_b0, p_b1), (sem_p0, sem_p1), denom_sp,
                        sid, iota16)
        plsc.subcore_barrier()
        pltpu.sync_copy(denom_sp, denom_v)

        _sc_aggregate(h3_hbm, agg_sp, denom_v, s_score_v, d_score_v,
                      src3_hbm, dst3_hbm, src3h, dst3h,
                      (g0, g1), (s0, s1), (sem_g0, sem_g1),
                      (sem_s0, sem_s1), OUT, sid, iota16,
                      (cid * HNCH,))

        plsc.subcore_barrier()
        _spmem_to_hbm_rows(agg_sp, part_hbm.at[cid], g0, sid)

    return k


def _make_sc_classifier():
    scratch = [
        pltpu.VMEM((N,), _f32),
        pltpu.VMEM((N,), _f32),
        pltpu.VMEM((ELCH, CHUNK), _i32),
        pltpu.VMEM((ELCH, CHUNK), _i32),
        pltpu.VMEM((ELCH, CHUNK), _f32),
    ]

    @functools.partial(
        pl.kernel,
        out_type=jax.ShapeDtypeStruct((NC * NS, ELCH, CHUNK), _f32),
        mesh=_mesh(),
        scratch_types=scratch,
        compiler_params=_sc_params(),
    )
    def k(u_hbm, v_hbm, s3_hbm, d3_hbm, pred_hbm, u_v, v_v, s_v, d_v, out_v):
        cid = lax.axis_index("c")
        sid = lax.axis_index("s")
        wid = sid * NC + cid

        pltpu.sync_copy(u_hbm, u_v)
        pltpu.sync_copy(v_hbm, v_v)
        pltpu.sync_copy(s3_hbm.at[wid], s_v)
        pltpu.sync_copy(d3_hbm.at[wid], d_v)

        def chunk(c, carry):
            for g in range(8):
                s_i = s_v[c, pl.ds(g * 16, 16)]
                d_i = d_v[c, pl.ds(g * 16, 16)]
                u = plsc.load_gather(u_v, [s_i])
                v = plsc.load_gather(v_v, [d_i])
                z = u + v
                out_v[c, pl.ds(g * 16, 16)] = 1.0 / (1.0 + jnp.exp(-z))
            return carry

        lax.fori_loop(0, ELCH, chunk, 0)
        pltpu.sync_copy(out_v, pred_hbm.at[wid])

    return k


BLK = 1000
GRID = N // BLK


def _tc_layer1(x, W1, A1):
    def body(x_ref, w_ref, a_ref, q0_ref, q1_ref, q2_ref, q3_ref, sc_ref):
        h = jnp.dot(x_ref[...], w_ref[...], preferred_element_type=_f32)
        for i, q in enumerate((q0_ref, q1_ref, q2_ref, q3_ref)):
            q[...] = h[:, i * QW:(i + 1) * QW]
        sc_ref[...] = jnp.dot(h, a_ref[...], preferred_element_type=_f32)

    qspec = pl.BlockSpec((BLK, QW), lambda i: (i, 0))
    return pl.pallas_call(
        body,
        grid=(GRID,),
        in_specs=[
            pl.BlockSpec((BLK, D), lambda i: (i, 0)),
            pl.BlockSpec((D, HID), lambda i: (0, 0)),
            pl.BlockSpec((HID, 2), lambda i: (0, 0)),
        ],
        out_specs=[qspec, qspec, qspec, qspec,
                   pl.BlockSpec((BLK, 2), lambda i: (i, 0))],
        out_shape=[jax.ShapeDtypeStruct((N, QW), _f32) for _ in range(4)]
        + [jax.ShapeDtypeStruct((N, 2), _f32)],
    )(x, W1, A1)


def _tc_layer_mid(quarters, b4, W, A, n_out, split):
    def body(q0_ref, q1_ref, q2_ref, q3_ref, b_ref, w_ref, a_ref, *outs):
        w = w_ref[...]
        h = None
        for i, q in enumerate((q0_ref, q1_ref, q2_ref, q3_ref)):
            xq = jax.nn.relu(q[...] + b_ref[i, :])
            part = jnp.dot(xq, w[i * QW:(i + 1) * QW, :],
                           preferred_element_type=_f32)
            h = part if h is None else h + part
        if split:
            for i in range(4):
                outs[i][...] = h[:, i * QW:(i + 1) * QW]
        else:
            outs[0][...] = h
        outs[-1][...] = jnp.dot(h, a_ref[...], preferred_element_type=_f32)

    qspec = pl.BlockSpec((BLK, QW), lambda i: (i, 0))
    scspec = pl.BlockSpec((BLK, 2), lambda i: (i, 0))
    if split:
        out_specs = [qspec, qspec, qspec, qspec, scspec]
        out_shape = [jax.ShapeDtypeStruct((N, QW), _f32) for _ in range(4)] \
            + [jax.ShapeDtypeStruct((N, 2), _f32)]
    else:
        out_specs = [pl.BlockSpec((BLK, n_out), lambda i: (i, 0)), scspec]
        out_shape = [jax.ShapeDtypeStruct((N, n_out), _f32),
                     jax.ShapeDtypeStruct((N, 2), _f32)]

    return pl.pallas_call(
        body,
        grid=(GRID,),
        in_specs=[qspec, qspec, qspec, qspec,
                  pl.BlockSpec((4, QW), lambda i: (0, 0)),
                  pl.BlockSpec((HID, n_out), lambda i: (0, 0)),
                  pl.BlockSpec((n_out, 2), lambda i: (0, 0))],
        out_specs=out_specs,
        out_shape=out_shape,
    )(*quarters, b4, W, A)


def _tc_emb_uv(parts, b3_2d, Wuv, buv_2d):
    def body(p_ref, b_ref, w_ref, bu_ref, emb_ref, uv_ref):
        emb = p_ref[0] + p_ref[1] + b_ref[0, :]
        emb_ref[...] = emb
        uv_ref[...] = (jnp.dot(emb, w_ref[...], preferred_element_type=_f32)
                       + bu_ref[0, :])

    return pl.pallas_call(
        body,
        grid=(GRID,),
        in_specs=[
            pl.BlockSpec((NC, BLK, OUT), lambda i: (0, i, 0)),
            pl.BlockSpec((1, OUT), lambda i: (0, 0)),
            pl.BlockSpec((OUT, 2), lambda i: (0, 0)),
            pl.BlockSpec((1, 2), lambda i: (0, 0)),
        ],
        out_specs=[
            pl.BlockSpec((BLK, OUT), lambda i: (i, 0)),
            pl.BlockSpec((BLK, 2), lambda i: (i, 0)),
        ],
        out_shape=[
            jax.ShapeDtypeStruct((N, OUT), _f32),
            jax.ShapeDtypeStruct((N, 2), _f32),
        ],
    )(parts, b3_2d, Wuv, buv_2d)


_sc_agg_wide = functools.lru_cache(maxsize=None)(_make_sc_agg_wide)
_sc_agg_small = functools.lru_cache(maxsize=None)(_make_sc_agg_small)
_sc_classifier = functools.lru_cache(maxsize=None)(_make_sc_classifier)


def kernel(x, edge_index, edge_label_index, W1, as1, ad1, b1, W2, as2, ad2,
           b2, W3, as3, ad3, b3, Wc, bc):
    src = edge_index[0].astype(_i32)
    dst = edge_index[1].astype(_i32)
    pad = jnp.zeros((EPAD - E,), _i32)
    src3 = jnp.concatenate([src, pad]).reshape(NS, NCH, CHUNK)
    dst3 = jnp.concatenate([dst, pad]).reshape(NS, NCH, CHUNK)

    elpad = jnp.zeros((ELPAD - EL,), _i32)
    s3 = jnp.concatenate([edge_label_index[0].astype(_i32), elpad])
    d3 = jnp.concatenate([edge_label_index[1].astype(_i32), elpad])
    s3 = s3.reshape(NC * NS, ELCH, CHUNK)
    d3 = d3.reshape(NC * NS, ELCH, CHUNK)

    A1 = jnp.stack([as1, ad1], axis=1)
    A2 = jnp.stack([as2, ad2], axis=1)
    A3 = jnp.stack([as3, ad3], axis=1)
    b1_4 = b1.reshape(4, QW)
    b2_4 = b2.reshape(4, QW)
    b3_2d = b3.reshape(1, OUT)
    Wuv = Wc.reshape(2, OUT).T
    buv_2d = jnp.stack([bc[0], jnp.zeros((), _f32)]).reshape(1, 2)

    q0, q1, q2, q3, sc1 = _tc_layer1(x, W1, A1)
    a = _sc_agg_wide()(q0, q1, q2, q3, sc1[:, 0], sc1[:, 1], src3, dst3)

    *q2s, sc2 = _tc_layer_mid(a, b1_4, W2, A2, HID, True)
    a2 = _sc_agg_wide()(*q2s, sc2[:, 0], sc2[:, 1], src3, dst3)

    h3, sc3 = _tc_layer_mid(a2, b2_4, W3, A3, OUT, False)
    parts = _sc_agg_small()(h3, sc3[:, 0], sc3[:, 1], src3, dst3)

    emb, uv = _tc_emb_uv(parts, b3_2d, Wuv, buv_2d)
    pred3 = _sc_classifier()(uv[:, 0], uv[:, 1], s3, d3)
    pred = pred3.reshape(-1)[:EL]
    return pred, emb


# speedup vs baseline: 1.3901x; 1.0003x over previous
import functools

import jax
import jax.numpy as jnp
from jax import lax
from jax.experimental import pallas as pl
from jax.experimental.pallas import tpu as pltpu
from jax.experimental.pallas import tpu_sc as plsc

N = 10000
E = 320000
EL = 100000
D = 128
HID = 256
QW = 64
OUT = 16

NS = 16
NC = 2
CHUNK = 128
EPT = 20480
NCH = EPT // CHUNK
HNCH = NCH // 2
EPAD = NS * EPT
ELCH = 25
ELPAD = NC * NS * ELCH * CHUNK
ROWS_PT = 640
ROWS_LAST = N - 15 * ROWS_PT

_f32 = jnp.float32
_i32 = jnp.int32


def _mesh():
    return plsc.VectorSubcoreMesh(core_axis_name="c", subcore_axis_name="s")


def _sc_params():
    return pltpu.CompilerParams(needs_layout_passes=False,
                                use_tc_tiling_on_sc=False)


def _zero_vmem_2d(buf, width):
    zero16 = lax.full((16,), 0.0, _f32)

    def row(r, carry):
        for cc in range(width // 16):
            buf[r, pl.ds(cc * 16, 16)] = zero16
        return carry

    lax.fori_loop(0, CHUNK, row, 0)


def _zero_spmem_rows(zbuf, agg_sp, sid):
    @pl.when(sid < 15)
    def _():
        for k in range(5):
            pltpu.sync_copy(zbuf,
                            agg_sp.at[pl.ds(sid * ROWS_PT + k * CHUNK,
                                            CHUNK)])

    @pl.when(sid == 15)
    def _():
        for k in range(3):
            pltpu.sync_copy(zbuf,
                            agg_sp.at[pl.ds(15 * ROWS_PT + k * CHUNK,
                                            CHUNK)])
        pltpu.sync_copy(zbuf.at[pl.ds(0, 16)],
                        agg_sp.at[pl.ds(15 * ROWS_PT + 3 * CHUNK, 16)])


def _spmem_to_hbm_rows(agg_sp, out_hbm, bounce, sid):
    @pl.when(sid < 15)
    def _():
        for k in range(5):
            rows = pl.ds(sid * ROWS_PT + k * CHUNK, CHUNK)
            pltpu.sync_copy(agg_sp.at[rows], bounce)
            pltpu.sync_copy(bounce, out_hbm.at[rows])

    @pl.when(sid == 15)
    def _():
        for k in range(3):
            rows = pl.ds(15 * ROWS_PT + k * CHUNK, CHUNK)
            pltpu.sync_copy(agg_sp.at[rows], bounce)
            pltpu.sync_copy(bounce, out_hbm.at[rows])
        rows = pl.ds(15 * ROWS_PT + 3 * CHUNK, 16)
        pltpu.sync_copy(agg_sp.at[rows], bounce.at[pl.ds(0, 16)])
        pltpu.sync_copy(bounce.at[pl.ds(0, 16)], out_hbm.at[rows])


def _zero_denom(p_b, denom_sp, sid):
    zero16 = lax.full((16,), 0.0, _f32)
    for i in range(8):
        p_b[pl.ds(i * 16, 16)] = zero16

    @pl.when(sid < 10)
    def _():
        for k in range(8):
            pltpu.sync_copy(p_b.at[pl.ds(0, 120)],
                            denom_sp.at[pl.ds(sid * 1000 + k * 120, 120)])
        pltpu.sync_copy(p_b.at[pl.ds(0, 40)],
                        denom_sp.at[pl.ds(sid * 1000 + 960, 40)])


def _edge_p(s_score_v, d_score_v, src3h, dst3h, sid, cl, cg, g, iota16):
    s_i = src3h[cl, pl.ds(g * 16, 16)]
    d_i = dst3h[cl, pl.ds(g * 16, 16)]
    s_v = plsc.load_gather(s_score_v, [s_i])
    d_v = plsc.load_gather(d_score_v, [d_i])
    l = s_v + d_v
    l = jnp.where(l > 0, l, 0.2 * l)
    p = jnp.exp(l)
    gid = sid * EPT + cg * CHUNK + g * 16 + iota16
    return jnp.where(gid < E, p, 0.0), d_i


def _load_idx_half(src3_hbm, dst3_hbm, sid, h0, src3h, dst3h):
    pltpu.sync_copy(src3_hbm.at[sid, pl.ds(h0, HNCH)], src3h)
    pltpu.sync_copy(dst3_hbm.at[sid, pl.ds(h0, HNCH)], dst3h)


def _sc_denom_phase(s_score_v, d_score_v, src3_hbm, dst3_hbm, src3h, dst3h,
                    p_bufs, p_sems, denom_sp, sid, iota16):
    zero16 = lax.full((16,), 0.0, _f32)
    for h0 in (0, HNCH):
        _load_idx_half(src3_hbm, dst3_hbm, sid, h0, src3h, dst3h)
        for par in (0, 1):
            for g in range(8):
                p_bufs[par][pl.ds(g * 16, 16)] = zero16
            pltpu.async_copy(p_bufs[par], denom_sp.at[dst3h.at[par]],
                             p_sems[par], add=True)

        def pair(i, carry):
            for par in (0, 1):
                cl = 2 * i + par
                pb = p_bufs[par]
                pltpu.make_async_copy(pb, denom_sp.at[dst3h.at[cl]],
                                      p_sems[par]).wait()
                for g in range(8):
                    p, _ = _edge_p(s_score_v, d_score_v, src3h, dst3h, sid,
                                   cl, h0 + cl, g, iota16)
                    pb[pl.ds(g * 16, 16)] = p
                pltpu.async_copy(pb, denom_sp.at[dst3h.at[cl]], p_sems[par],
                                 add=True)
            return carry

        lax.fori_loop(0, HNCH // 2, pair, 0)
        for par in (0, 1):
            pltpu.make_async_copy(p_bufs[par], denom_sp.at[dst3h.at[par]],
                                  p_sems[par]).wait()


def _sc_aggregate(h_hbm, agg_sp, denom_v, s_score_v, d_score_v,
                  src3_hbm, dst3_hbm, src3h, dst3h, g_bufs,
                  s_bufs, g_sems, s_sems, width, sid, iota16,
                  halves):
    nsub = width // 16
    lane_consts = [lax.full((16,), j, _i32) for j in range(16)]

    def scale_chunk(cl, cg, par):
        g_buf, s_buf = g_bufs[par], s_bufs[par]

        def grp(g, carry2):
            p, d_i = _edge_p(s_score_v, d_score_v, src3h, dst3h, sid, cl,
                             cg, g, iota16)
            den16 = plsc.load_gather(denom_v, [d_i])
            alpha16 = p / (den16 + 1e-16)
            for j in range(16):
                a_b = alpha16.at[lane_consts[j]].get(
                    mode="promise_in_bounds")
                r = g * 16 + j
                for cc in range(nsub):
                    s_buf[r, pl.ds(cc * 16, 16)] = (
                        g_buf[r, pl.ds(cc * 16, 16)] * a_b)
            return carry2

        lax.fori_loop(0, 8, grp, 0, unroll=4)

    def issue_gather(cl, par):
        pltpu.async_copy(h_hbm.at[src3h.at[cl]], g_bufs[par], g_sems[par])

    def wait_gather(cl, par):
        pltpu.make_async_copy(h_hbm.at[src3h.at[cl]], g_bufs[par],
                              g_sems[par]).wait()

    def issue_scatter(cl, par):
        pltpu.async_copy(s_bufs[par], agg_sp.at[dst3h.at[cl]], s_sems[par],
                         add=True)

    def wait_scatter(cl, par):
        pltpu.make_async_copy(s_bufs[par], agg_sp.at[dst3h.at[cl]],
                              s_sems[par]).wait()

    for h0 in halves:
        _load_idx_half(src3_hbm, dst3_hbm, sid, h0, src3h, dst3h)
        _zero_vmem_2d(s_bufs[0], width)
        _zero_vmem_2d(s_bufs[1], width)
        issue_scatter(0, 0)
        issue_scatter(1, 1)
        issue_gather(0, 0)

        def pair(i, carry):
            for par in (0, 1):
                cl = 2 * i + par
                issue_gather(jnp.minimum(cl + 1, HNCH - 1), 1 - par)
                wait_gather(cl, par)
                wait_scatter(cl, par)
                scale_chunk(cl, h0 + cl, par)
                issue_scatter(cl, par)
            return carry

        lax.fori_loop(0, HNCH // 2, pair, 0)
        wait_gather(HNCH - 1, 0)
        wait_scatter(HNCH - 2, 0)
        wait_scatter(HNCH - 1, 1)


def _make_sc_agg_wide():
    scratch = [
        pltpu.VMEM((N,), _f32),
        pltpu.VMEM((N,), _f32),
        pltpu.VMEM((HNCH, CHUNK), _i32),
        pltpu.VMEM((HNCH, CHUNK), _i32),
        pltpu.VMEM((CHUNK,), _f32),
        pltpu.VMEM((CHUNK,), _f32),
        pltpu.VMEM((N,), _f32),
        pltpu.VMEM((CHUNK, QW), _f32),
        pltpu.VMEM((CHUNK, QW), _f32),
        pltpu.VMEM((CHUNK, QW), _f32),
        pltpu.VMEM((CHUNK, QW), _f32),
    ] + [pltpu.SemaphoreType.DMA] * 6 + [
        pltpu.VMEM_SHARED((N,), _f32),
        pltpu.VMEM_SHARED((N, QW), _f32),
    ]

    @functools.partial(
        pl.kernel,
        out_type=tuple(jax.ShapeDtypeStruct((N, QW), _f32)
                       for _ in range(4)),
        mesh=_mesh(),
        scratch_types=scratch,
        compiler_params=_sc_params(),
    )
    def k(h0_hbm, h1_hbm, h2_hbm, h3_hbm, s_hbm, d_hbm, src3_hbm, dst3_hbm,
          a0_hbm, a1_hbm, a2_hbm, a3_hbm,
          s_score_v, d_score_v, src3h, dst3h, p_b0, p_b1, denom_v,
          g0, g1, s0, s1, sem_p0, sem_p1, sem_g0, sem_g1, sem_s0, sem_s1,
          denom_sp, agg_sp):
        cid = lax.axis_index("c")
        sid = lax.axis_index("s")
        iota16 = lax.iota(_i32, 16)

        pltpu.sync_copy(s_hbm, s_score_v)
        pltpu.sync_copy(d_hbm, d_score_v)

        _zero_vmem_2d(s0, QW)
        _zero_spmem_rows(s0, agg_sp, sid)
        _zero_denom(p_b0, denom_sp, sid)
        plsc.subcore_barrier()

        _sc_denom_phase(s_score_v, d_score_v, src3_hbm, dst3_hbm, src3h,
                        dst3h, (p_b0, p_b1), (sem_p0, sem_p1), denom_sp,
                        sid, iota16)
        plsc.subcore_barrier()
        pltpu.sync_copy(denom_sp, denom_v)

        def one_pass(h_hbm, out_hbm):
            _sc_aggregate(h_hbm, agg_sp, denom_v, s_score_v, d_score_v,
                          src3_hbm, dst3_hbm, src3h, dst3h,
                          (g0, g1), (s0, s1), (sem_g0, sem_g1),
                          (sem_s0, sem_s1), QW, sid, iota16, (0, HNCH))
            plsc.subcore_barrier()
            _spmem_to_hbm_rows(agg_sp, out_hbm, g0, sid)

        def second_pass_prep():
            plsc.subcore_barrier()
            _zero_vmem_2d(s0, QW)
            _zero_spmem_rows(s0, agg_sp, sid)
            plsc.subcore_barrier()

        @pl.when(cid == 0)
        def _():
            one_pass(h0_hbm, a0_hbm)
            second_pass_prep()
            one_pass(h1_hbm, a1_hbm)

        @pl.when(cid == 1)
        def _():
            one_pass(h2_hbm, a2_hbm)
            second_pass_prep()
            one_pass(h3_hbm, a3_hbm)

    return k


def _make_sc_agg_small():
    scratch = [
        pltpu.VMEM((N,), _f32),
        pltpu.VMEM((N,), _f32),
        pltpu.VMEM((HNCH, CHUNK), _i32),
        pltpu.VMEM((HNCH, CHUNK), _i32),
        pltpu.VMEM((CHUNK,), _f32),
        pltpu.VMEM((CHUNK,), _f32),
        pltpu.VMEM((N,), _f32),
        pltpu.VMEM((CHUNK, OUT), _f32),
        pltpu.VMEM((CHUNK, OUT), _f32),
        pltpu.VMEM((CHUNK, OUT), _f32),
        pltpu.VMEM((CHUNK, OUT), _f32),
    ] + [pltpu.SemaphoreType.DMA] * 6 + [
        pltpu.VMEM_SHARED((N,), _f32),
        pltpu.VMEM_SHARED((N, OUT), _f32),
    ]

    @functools.partial(
        pl.kernel,
        out_type=jax.ShapeDtypeStruct((NC, N, OUT), _f32),
        mesh=_mesh(),
        scratch_types=scratch,
        compiler_params=_sc_params(),
    )
    def k(h3_hbm, s_hbm, d_hbm, src3_hbm, dst3_hbm, part_hbm,
          s_score_v, d_score_v, src3h, dst3h, p_b0, p_b1, denom_v,
          g0, g1, s0, s1, sem_p0, sem_p1, sem_g0, sem_g1, sem_s0, sem_s1,
          denom_sp, agg_sp):
        cid = lax.axis_index("c")
        sid = lax.axis_index("s")
        iota16 = lax.iota(_i32, 16)

        pltpu.sync_copy(s_hbm, s_score_v)
        pltpu.sync_copy(d_hbm, d_score_v)

        _zero_vmem_2d(s0, OUT)
        _zero_spmem_rows(s0, agg_sp, sid)
        _zero_denom(p_b0, denom_sp, sid)
        plsc.subcore_barrier()

        _sc_denom_phase(s_score_v, d_score_v, src3_hbm, dst3_hbm, src3h,
                        dst3h, (p_b0, p_b1), (sem_p0, sem_p1), denom_sp,
                        sid, iota16)
        plsc.subcore_barrier()
        pltpu.sync_copy(denom_sp, denom_v)

        _sc_aggregate(h3_hbm, agg_sp, denom_v, s_score_v, d_score_v,
                      src3_hbm, dst3_hbm, src3h, dst3h,
                      (g0, g1), (s0, s1), (sem_g0, sem_g1),
                      (sem_s0, sem_s1), OUT, sid, iota16,
                      (cid * HNCH,))

        plsc.subcore_barrier()
        _spmem_to_hbm_rows(agg_sp, part_hbm.at[cid], g0, sid)

    return k


def _make_sc_classifier():
    scratch = [
        pltpu.VMEM((N,), _f32),
        pltpu.VMEM((N,), _f32),
        pltpu.VMEM((ELCH, CHUNK), _i32),
        pltpu.VMEM((ELCH, CHUNK), _i32),
        pltpu.VMEM((ELCH, CHUNK), _f32),
    ]

    @functools.partial(
        pl.kernel,
        out_type=jax.ShapeDtypeStruct((NC * NS, ELCH, CHUNK), _f32),
        mesh=_mesh(),
        scratch_types=scratch,
        compiler_params=_sc_params(),
    )
    def k(u_hbm, v_hbm, s3_hbm, d3_hbm, pred_hbm, u_v, v_v, s_v, d_v, out_v):
        cid = lax.axis_index("c")
        sid = lax.axis_index("s")
        wid = sid * NC + cid

        pltpu.sync_copy(u_hbm, u_v)
        pltpu.sync_copy(v_hbm, v_v)
        pltpu.sync_copy(s3_hbm.at[wid], s_v)
        pltpu.sync_copy(d3_hbm.at[wid], d_v)

        def chunk(c, carry):
            for g in range(8):
                s_i = s_v[c, pl.ds(g * 16, 16)]
                d_i = d_v[c, pl.ds(g * 16, 16)]
                u = plsc.load_gather(u_v, [s_i])
                v = plsc.load_gather(v_v, [d_i])
                z = u + v
                out_v[c, pl.ds(g * 16, 16)] = 1.0 / (1.0 + jnp.exp(-z))
            return carry

        lax.fori_loop(0, ELCH, chunk, 0)
        pltpu.sync_copy(out_v, pred_hbm.at[wid])

    return k


BLK = 1000
GRID = N // BLK


def _tc_layer1(x, W1, A1):
    def body(x_ref, w_ref, a_ref, q0_ref, q1_ref, q2_ref, q3_ref, sc_ref):
        h = jnp.dot(x_ref[...], w_ref[...], preferred_element_type=_f32)
        for i, q in enumerate((q0_ref, q1_ref, q2_ref, q3_ref)):
            q[...] = h[:, i * QW:(i + 1) * QW]
        sc_ref[...] = jnp.dot(h, a_ref[...], preferred_element_type=_f32)

    qspec = pl.BlockSpec((BLK, QW), lambda i: (i, 0))
    return pl.pallas_call(
        body,
        grid=(GRID,),
        in_specs=[
            pl.BlockSpec((BLK, D), lambda i: (i, 0)),
            pl.BlockSpec((D, HID), lambda i: (0, 0)),
            pl.BlockSpec((HID, 2), lambda i: (0, 0)),
        ],
        out_specs=[qspec, qspec, qspec, qspec,
                   pl.BlockSpec((BLK, 2), lambda i: (i, 0))],
        out_shape=[jax.ShapeDtypeStruct((N, QW), _f32) for _ in range(4)]
        + [jax.ShapeDtypeStruct((N, 2), _f32)],
    )(x, W1, A1)


def _tc_layer_mid(quarters, b4, W, A, n_out, split):
    def body(q0_ref, q1_ref, q2_ref, q3_ref, b_ref, w_ref, a_ref, *outs):
        w = w_ref[...]
        h = None
        for i, q in enumerate((q0_ref, q1_ref, q2_ref, q3_ref)):
            xq = jax.nn.relu(q[...] + b_ref[i, :])
            part = jnp.dot(xq, w[i * QW:(i + 1) * QW, :],
                           preferred_element_type=_f32)
            h = part if h is None else h + part
        if split:
            for i in range(4):
                outs[i][...] = h[:, i * QW:(i + 1) * QW]
        else:
            outs[0][...] = h
        outs[-1][...] = jnp.dot(h, a_ref[...], preferred_element_type=_f32)

    qspec = pl.BlockSpec((BLK, QW), lambda i: (i, 0))
    scspec = pl.BlockSpec((BLK, 2), lambda i: (i, 0))
    if split:
        out_specs = [qspec, qspec, qspec, qspec, scspec]
        out_shape = [jax.ShapeDtypeStruct((N, QW), _f32) for _ in range(4)] \
            + [jax.ShapeDtypeStruct((N, 2), _f32)]
    else:
        out_specs = [pl.BlockSpec((BLK, n_out), lambda i: (i, 0)), scspec]
        out_shape = [jax.ShapeDtypeStruct((N, n_out), _f32),
                     jax.ShapeDtypeStruct((N, 2), _f32)]

    return pl.pallas_call(
        body,
        grid=(GRID,),
        in_specs=[qspec, qspec, qspec, qspec,
                  pl.BlockSpec((4, QW), lambda i: (0, 0)),
                  pl.BlockSpec((HID, n_out), lambda i: (0, 0)),
                  pl.BlockSpec((n_out, 2), lambda i: (0, 0))],
        out_specs=out_specs,
        out_shape=out_shape,
    )(*quarters, b4, W, A)


def _tc_emb_uv(parts, b3_2d, Wuv, buv_2d):
    def body(p_ref, b_ref, w_ref, bu_ref, emb_ref, uv_ref):
        emb = p_ref[0] + p_ref[1] + b_ref[0, :]
        emb_ref[...] = emb
        uv_ref[...] = (jnp.dot(emb, w_ref[...], preferred_element_type=_f32)
                       + bu_ref[0, :])

    return pl.pallas_call(
        body,
        grid=(GRID,),
        in_specs=[
            pl.BlockSpec((NC, BLK, OUT), lambda i: (0, i, 0)),
            pl.BlockSpec((1, OUT), lambda i: (0, 0)),
            pl.BlockSpec((OUT, 2), lambda i: (0, 0)),
            pl.BlockSpec((1, 2), lambda i: (0, 0)),
        ],
        out_specs=[
            pl.BlockSpec((BLK, OUT), lambda i: (i, 0)),
            pl.BlockSpec((BLK, 2), lambda i: (i, 0)),
        ],
        out_shape=[
            jax.ShapeDtypeStruct((N, OUT), _f32),
            jax.ShapeDtypeStruct((N, 2), _f32),
        ],
    )(parts, b3_2d, Wuv, buv_2d)


_sc_agg_wide = functools.lru_cache(maxsize=None)(_make_sc_agg_wide)
_sc_agg_small = functools.lru_cache(maxsize=None)(_make_sc_agg_small)
_sc_classifier = functools.lru_cache(maxsize=None)(_make_sc_classifier)


def kernel(x, edge_index, edge_label_index, W1, as1, ad1, b1, W2, as2, ad2,
           b2, W3, as3, ad3, b3, Wc, bc):
    src = edge_index[0].astype(_i32)
    dst = edge_index[1].astype(_i32)
    pad = jnp.zeros((EPAD - E,), _i32)
    src3 = jnp.concatenate([src, pad]).reshape(NS, NCH, CHUNK)
    dst3 = jnp.concatenate([dst, pad]).reshape(NS, NCH, CHUNK)

    elpad = jnp.zeros((ELPAD - EL,), _i32)
    s3 = jnp.concatenate([edge_label_index[0].astype(_i32), elpad])
    d3 = jnp.concatenate([edge_label_index[1].astype(_i32), elpad])
    s3 = s3.reshape(NC * NS, ELCH, CHUNK)
    d3 = d3.reshape(NC * NS, ELCH, CHUNK)

    A1 = jnp.stack([as1, ad1], axis=1)
    A2 = jnp.stack([as2, ad2], axis=1)
    A3 = jnp.stack([as3, ad3], axis=1)
    b1_4 = b1.reshape(4, QW)
    b2_4 = b2.reshape(4, QW)
    b3_2d = b3.reshape(1, OUT)
    Wuv = Wc.reshape(2, OUT).T
    buv_2d = jnp.stack([bc[0], jnp.zeros((), _f32)]).reshape(1, 2)

    q0, q1, q2, q3, sc1 = _tc_layer1(x, W1, A1)
    a = _sc_agg_wide()(q0, q1, q2, q3, sc1[:, 0], sc1[:, 1], src3, dst3)

    *q2s, sc2 = _tc_layer_mid(a, b1_4, W2, A2, HID, True)
    a2 = _sc_agg_wide()(*q2s, sc2[:, 0], sc2[:, 1], src3, dst3)

    h3, sc3 = _tc_layer_mid(a2, b2_4, W3, A3, OUT, False)
    parts = _sc_agg_small()(h3, sc3[:, 0], sc3[:, 1], src3, dst3)

    emb, uv = _tc_emb_uv(parts, b3_2d, Wuv, buv_2d)
    pred3 = _sc_classifier()(uv[:, 0], uv[:, 1], s3, d3)
    pred = pred3.reshape(-1)[:EL]
    return pred, emb
